# baseline (speedup 1.0000x reference)
import functools

import jax
import jax.numpy as jnp
from jax import lax
from jax.experimental import pallas as pl
from jax.experimental.pallas import tpu as pltpu
from jax.experimental.pallas import tpu_sc as plsc

H = W = 14
HW = H * W
V = (2 * H - 1) * (2 * W - 1)
IO = 32
NJV = 13
VP = V


def _make_gather():
    mesh = plsc.VectorSubcoreMesh(core_axis_name="c", subcore_axis_name="s")

    @functools.partial(
        pl.kernel,
        mesh=mesh,
        out_type=jax.ShapeDtypeStruct((HW, IO, IO, HW), jnp.float32),
        compiler_params=pltpu.CompilerParams(needs_layout_passes=False),
        scratch_types=[
            pltpu.VMEM((HW, HW), jnp.int32),
            pltpu.VMEM((IO * V,), jnp.float32),
            pltpu.VMEM((3, 2, IO, HW), jnp.float32),
            pltpu.SemaphoreType.DMA,
        ],
    )
    def gather_kernel(tabt_hbm, idx_hbm, out_hbm, idx_v, tab_v, buf_v, sem):
        w = lax.axis_index("s") * 2 + lax.axis_index("c")

        pltpu.sync_copy(idx_hbm, idx_v)
        pltpu.sync_copy(tabt_hbm.at[w], tab_v)

        offs = [jv * 16 for jv in range(NJV - 1)] + [HW - 16]

        def drain_one():
            pltpu.make_async_copy(
                buf_v.at[0], out_hbm.at[pl.ds(0, 2), w], sem).wait()

        def body(ii, _):
            ph = lax.rem(ii, 3)

            @pl.when(ii >= 3)
            def _():
                drain_one()

            i = ii * 2
            ivs = [idx_v[i, pl.ds(joff, 16)] for joff in offs]
            ivs2 = [idx_v[i + 1, pl.ds(joff, 16)] for joff in offs]

            @plsc.parallel_loop(0, IO, unroll=2)
            def out_body(o):
                ov = jnp.full((16,), o * VP, jnp.int32)
                for jv, joff in enumerate(offs):
                    vals = plsc.load_gather(tab_v, [ov + ivs[jv]])
                    buf_v[ph, 0, o, pl.ds(joff, 16)] = vals
                    vals2 = plsc.load_gather(tab_v, [ov + ivs2[jv]])
                    buf_v[ph, 1, o, pl.ds(joff, 16)] = vals2

            pltpu.async_copy(buf_v.at[ph], out_hbm.at[pl.ds(i, 2), w], sem)
            return _

        lax.fori_loop(0, HW // 2, body, None)
        for _ in range(3):
            drain_one()

    return gather_kernel


_gather = _make_gather()


def kernel(unique_params, index_map):
    tabt = unique_params.transpose(1, 2, 0).reshape(IO, IO * V)
    out = _gather(tabt, index_map.astype(jnp.int32))
    return out.transpose(0, 3, 1, 2)


# speedup vs baseline: 1.3194x; 1.3194x over previous
import functools

import jax
import jax.numpy as jnp
from jax import lax
from jax.experimental import pallas as pl
from jax.experimental.pallas import tpu as pltpu
from jax.experimental.pallas import tpu_sc as plsc

H = W = 14
HW = H * W
V = (2 * H - 1) * (2 * W - 1)
IO = 32
NJV = 13
VP = V


def _make_gather():
    mesh = plsc.VectorSubcoreMesh(core_axis_name="c", subcore_axis_name="s")

    @functools.partial(
        pl.kernel,
        mesh=mesh,
        out_type=jax.ShapeDtypeStruct((HW, IO, IO, HW), jnp.float32),
        compiler_params=pltpu.CompilerParams(needs_layout_passes=False),
        scratch_types=[
            pltpu.VMEM((HW, HW), jnp.int32),
            pltpu.VMEM((IO * V,), jnp.float32),
            pltpu.VMEM((4, IO, HW), jnp.float32),
            pltpu.SemaphoreType.DMA,
        ],
    )
    def gather_kernel(tabt_hbm, idx_hbm, out_hbm, idx_v, tab_v, buf_v, sem):
        w = lax.axis_index("s") * 2 + lax.axis_index("c")

        pltpu.sync_copy(idx_hbm, idx_v)
        pltpu.sync_copy(tabt_hbm.at[w], tab_v)

        offs = [jv * 16 for jv in range(NJV - 1)] + [HW - 16]

        def drain_one():
            pltpu.make_async_copy(
                buf_v.at[0], out_hbm.at[0, w], sem).wait()

        def body(i, _):
            ph = lax.rem(i, 4)

            @pl.when(i >= 4)
            def _():
                drain_one()

            ivs = [idx_v[i, pl.ds(joff, 16)] for joff in offs]

            @plsc.parallel_loop(0, IO, unroll=4)
            def out_body(o):
                ov = jnp.full((16,), o * VP, jnp.int32)
                for jv, joff in enumerate(offs):
                    vals = plsc.load_gather(tab_v, [ov + ivs[jv]])
                    buf_v[ph, o, pl.ds(joff, 16)] = vals

            pltpu.async_copy(buf_v.at[ph], out_hbm.at[i, w], sem)
            return _

        lax.fori_loop(0, HW, body, None)
        for _ in range(4):
            drain_one()

    return gather_kernel


_gather = _make_gather()


def kernel(unique_params, index_map):
    tabt = unique_params.transpose(1, 2, 0).reshape(IO, IO * V)
    out = _gather(tabt, index_map.astype(jnp.int32))
    return out.transpose(0, 3, 1, 2)
